# baseline (speedup 1.0000x reference)
import functools

import jax
import jax.numpy as jnp
from jax import lax
from jax.experimental import pallas as pl
from jax.experimental.pallas import tpu as pltpu
from jax.experimental.pallas import tpu_sc as plsc

B, T, D = 4096, 77, 768
NW = 32
BPW = B // NW
CH = 64
NSUB = BPW // CH
NG = T * NSUB


def _body(xT, tok, pos, out, idx_all, prow, rowsA, rowsB,
          psem, gsemA, gsemB, ssemA, ssemB):
    wid = lax.axis_index("s") * 2 + lax.axis_index("c")
    b0 = wid * BPW

    pltpu.sync_copy(xT.at[:, pl.ds(b0, BPW)], idx_all)

    def pos_load(t):
        sel = lax.rem(t, 3)
        return pltpu.make_async_copy(
            pos.at[pl.ds(t, 1)], prow.at[pl.ds(sel, 1)], psem.at[sel])

    def idx_ref(g):
        return idx_all.at[g // NSUB, pl.ds((g % NSUB) * CH, CH)]

    def out_ref(g):
        return out.at[pl.ds(b0 + (g % NSUB) * CH, CH),
                      pl.ds(g // NSUB, 1)]

    def add_pos(g, buf):
        sel = lax.rem(g // NSUB, 3)
        for h in range(2):
            pv = tuple(prow[sel, pl.ds(h * 384 + j * 16, 16)]
                       for j in range(24))

            def r_body(r, carry):
                for j in range(24):
                    plsc.addupdate(buf.at[r, 0, pl.ds(h * 384 + j * 16, 16)],
                                   carry[j])
                return carry

            lax.fori_loop(0, CH, r_body, pv)

    bufs = ((rowsA, gsemA, ssemA), (rowsB, gsemB, ssemB))
    pos_load(0).start()
    pos_load(0).wait()
    pos_load(1).start()
    pltpu.async_copy(tok.at[idx_ref(0)], rowsA, gsemA)

    def g2_body(g2, _):
        for bpar in range(2):
            g = g2 * 2 + bpar
            cur_buf, cur_g, cur_s = bufs[bpar]
            nxt_buf, nxt_g, nxt_s = bufs[1 - bpar]

            if bpar == 0:
                t = g // NSUB

                @pl.when(t >= 1)
                def _():
                    pos_load(t).wait()

                @pl.when(t + 2 < T)
                def _():
                    pos_load(t + 2).start()

            @pl.when(g >= 1)
            def _():
                pltpu.make_async_copy(nxt_buf, out_ref(g - 1), nxt_s).wait()

            @pl.when(g + 1 < NG)
            def _():
                pltpu.async_copy(tok.at[idx_ref(g + 1)], nxt_buf, nxt_g)

            pltpu.make_async_copy(tok.at[idx_ref(g)], cur_buf, cur_g).wait()
            add_pos(g, cur_buf)
            pltpu.async_copy(cur_buf, out_ref(g), cur_s)
        return 0

    lax.fori_loop(0, NG // 2, g2_body, 0)
    pltpu.make_async_copy(rowsB, out_ref(NG - 1), ssemB).wait()


@jax.jit
def kernel(x, tok_embed, pos_embed):
    xT = x.astype(jnp.int32).T
    tok3 = tok_embed.reshape(tok_embed.shape[0], 1, D)
    mesh = plsc.VectorSubcoreMesh(core_axis_name="c", subcore_axis_name="s")
    k = functools.partial(
        pl.kernel,
        mesh=mesh,
        out_type=jax.ShapeDtypeStruct((B, T, D), jnp.float32),
        scratch_types=[
            pltpu.VMEM((T, BPW), jnp.int32),
            pltpu.VMEM((3, D), jnp.float32),
            pltpu.VMEM((CH, 1, D), jnp.float32),
            pltpu.VMEM((CH, 1, D), jnp.float32),
            pltpu.SemaphoreType.DMA((3,)),
            pltpu.SemaphoreType.DMA,
            pltpu.SemaphoreType.DMA,
            pltpu.SemaphoreType.DMA,
            pltpu.SemaphoreType.DMA,
        ],
    )(_body)
    return k(xT, tok3, pos_embed)
